# baseline (speedup 1.0000x reference)
import jax
import jax.numpy as jnp
from jax import lax
from jax.experimental import pallas as pl
from jax.experimental.pallas import tpu as pltpu
from jax.experimental.pallas import tpu_sc as plsc

B = 16384
E = 64
F = 8

_NC = 2
_NS = 16
_NW = _NC * _NS
_BPW = B // _NW
_NCHUNK = 8
_CH = _BPW // _NCHUNK

_SMALL_OFFS = (0, 7, 11, 14, 16, 21, 23)
_SMALL_ROWS = 36


def _emb_body(id_h, age_h, pvalue_h, shop_h, occu_h, city_h, gender_h, cms_h,
              w_id_h, w_small_h, out_h,
              idx_v, ws_v, idbuf_v, asm_v, spidx_sh, sidx_m,
              isem, ssem, gsems, wsems):
    cid = lax.axis_index("c")
    sid = lax.axis_index("s")
    wid = sid * _NC + cid
    base = wid * _BPW
    idx_hbm = (age_h, pvalue_h, shop_h, occu_h, city_h, gender_h, cms_h)

    pltpu.sync_copy(id_h.at[pl.ds(base, _BPW)], idx_v.at[F - 1])

    gd = [pltpu.async_copy(w_id_h.at[idx_v.at[F - 1, pl.ds(k * _CH, _CH)]],
                           idbuf_v.at[k], gsems.at[k])
          for k in range(_NCHUNK)]

    icopies = [pltpu.async_copy(idx_hbm[f].at[pl.ds(base, _BPW)],
                                idx_v.at[f], isem) for f in range(F - 1)]
    scopy = pltpu.async_copy(w_small_h, ws_v, ssem)
    for c in icopies:
        c.wait()
    scopy.wait()
    bcopies = [pltpu.async_copy(idx_v.at[f, pl.ds(k * _CH, _CH)],
                                spidx_sh.at[sid, k, f], isem)
               for k in range(_NCHUNK) for f in range(F - 1)]
    for c in bcopies:
        c.wait()

    wd = [None] * _NCHUNK
    for k in range(_NCHUNK):
        p = k % 2
        pltpu.sync_copy(spidx_sh.at[sid, k], sidx_m)
        if k >= 2:
            wd[k - 2].wait()
        gd[k].wait()
        asm_p = asm_v.at[p]

        def row_body(i, _, asm_p=asm_p, k=k):
            tr = i // 8
            r8 = lax.rem(i, 8)
            for c in range(E // 16):
                asm_p[tr, 0, r8, pl.ds(c * 16, 16)] = \
                    idbuf_v[k, i, pl.ds(c * 16, 16)]
            for f in range(F - 1):
                s = lax.min(lax.max(sidx_m[f, i], 0),
                            _SMALL_ROWS - 1 - _SMALL_OFFS[f]) + _SMALL_OFFS[f]
                col0 = ((f + 1) % 2) * 64
                for c in range(E // 16):
                    asm_p[tr, (f + 1) // 2, r8, pl.ds(col0 + c * 16, 16)] = \
                        ws_v[s, pl.ds(c * 16, 16)]
            return _

        lax.fori_loop(0, _CH, row_body, None)
        wd[k] = pltpu.async_copy(asm_p,
                                 out_h.at[pl.ds((base + k * _CH) // 8,
                                                _CH // 8)],
                                 wsems.at[p])
    wd[_NCHUNK - 2].wait()
    wd[_NCHUNK - 1].wait()


_emb = pl.kernel(
    _emb_body,
    mesh=plsc.VectorSubcoreMesh(core_axis_name="c", subcore_axis_name="s"),
    out_type=jax.ShapeDtypeStruct((B // 8, 4, 8, 128), jnp.float32),
    scratch_types=[
        pltpu.VMEM((F, _BPW), jnp.int32),
        pltpu.VMEM((_SMALL_ROWS, E), jnp.float32),
        pltpu.VMEM((_NCHUNK, _CH, E), jnp.float32),
        pltpu.VMEM((2, _CH // 8, 4, 8, 128), jnp.float32),
        pltpu.VMEM_SHARED((_NS, _NCHUNK, F - 1, _CH), jnp.int32),
        pltpu.SMEM((F - 1, _CH), jnp.int32),
        pltpu.SemaphoreType.DMA,
        pltpu.SemaphoreType.DMA,
        pltpu.SemaphoreType.DMA((_NCHUNK,)),
        pltpu.SemaphoreType.DMA((2,)),
    ],
    compiler_params=pltpu.CompilerParams(use_tc_tiling_on_sc=False),
)


@jax.jit
def kernel(id, age, pvalue, shop, occu, city, gender, cms,
           W_id, W_age, W_pvalue, W_shop, W_occu, W_city, W_gender, W_cms):
    w_small = jnp.concatenate(
        [W_age, W_pvalue, W_shop, W_occu, W_city, W_gender, W_cms], axis=0)
    out = _emb(id, age, pvalue, shop, occu, city, gender, cms, W_id, w_small)
    return out.transpose(0, 2, 1, 3).reshape(B, F, E)


# speedup vs baseline: 1.2821x; 1.2821x over previous
import jax
import jax.numpy as jnp
from jax import lax
from jax.experimental import pallas as pl
from jax.experimental.pallas import tpu as pltpu
from jax.experimental.pallas import tpu_sc as plsc

B = 16384
E = 64
F = 8

_NC = 2
_NS = 16
_NW = _NC * _NS
_BPW = B // _NW
_NCHUNK = 8
_CH = _BPW // _NCHUNK

_SMALL_OFFS = (0, 7, 11, 14, 16, 21, 23)
_SMALL_ROWS = 36


def _emb_body(id_h, age_h, pvalue_h, shop_h, occu_h, city_h, gender_h, cms_h,
              w_id_h, w_small_h, out_h,
              idx_v, ws_v, idbuf_v, asm_v, spidx_sh, sidx_m,
              isem, ssem, gsems, wsems):
    cid = lax.axis_index("c")
    sid = lax.axis_index("s")
    wid = sid * _NC + cid
    base = wid * _BPW
    idx_hbm = (age_h, pvalue_h, shop_h, occu_h, city_h, gender_h, cms_h)

    pltpu.sync_copy(id_h.at[pl.ds(base, _BPW)], idx_v.at[F - 1])

    gd = [pltpu.async_copy(w_id_h.at[idx_v.at[F - 1, pl.ds(k * _CH, _CH)]],
                           idbuf_v.at[k], gsems.at[k])
          for k in range(_NCHUNK)]

    icopies = [pltpu.async_copy(idx_hbm[f].at[pl.ds(base, _BPW)],
                                idx_v.at[f], isem) for f in range(F - 1)]
    scopy = pltpu.async_copy(w_small_h, ws_v, ssem)
    for c in icopies:
        c.wait()
    scopy.wait()
    bcopies = [pltpu.async_copy(idx_v.at[f, pl.ds(k * _CH, _CH)],
                                spidx_sh.at[sid, k, f], isem)
               for k in range(_NCHUNK) for f in range(F - 1)]
    for c in bcopies:
        c.wait()

    wd = [None] * _NCHUNK
    for k in range(_NCHUNK):
        p = k % 2
        pltpu.sync_copy(spidx_sh.at[sid, k], sidx_m)
        if k >= 2:
            wd[k - 2].wait()
        gd[k].wait()
        asm_p = asm_v.at[p]

        def row_body(i, _, asm_p=asm_p, k=k):
            for c in range(E // 16):
                asm_p[i, pl.ds(c * 16, 16)] = idbuf_v[k, i, pl.ds(c * 16, 16)]
            for f in range(F - 1):
                s = lax.min(lax.max(sidx_m[f, i], 0),
                            _SMALL_ROWS - 1 - _SMALL_OFFS[f]) + _SMALL_OFFS[f]
                for c in range(E // 16):
                    asm_p[i, pl.ds((f + 1) * E + c * 16, 16)] = \
                        ws_v[s, pl.ds(c * 16, 16)]
            return _

        lax.fori_loop(0, _CH, row_body, None)
        wd[k] = pltpu.async_copy(asm_p,
                                 out_h.at[pl.ds(base + k * _CH, _CH)],
                                 wsems.at[p])
    wd[_NCHUNK - 2].wait()
    wd[_NCHUNK - 1].wait()


_emb = pl.kernel(
    _emb_body,
    mesh=plsc.VectorSubcoreMesh(core_axis_name="c", subcore_axis_name="s"),
    out_type=jax.ShapeDtypeStruct((B, F * E), jnp.float32),
    scratch_types=[
        pltpu.VMEM((F, _BPW), jnp.int32),
        pltpu.VMEM((_SMALL_ROWS, E), jnp.float32),
        pltpu.VMEM((_NCHUNK, _CH, E), jnp.float32),
        pltpu.VMEM((2, _CH, F * E), jnp.float32),
        pltpu.VMEM_SHARED((_NS, _NCHUNK, F - 1, _CH), jnp.int32),
        pltpu.SMEM((F - 1, _CH), jnp.int32),
        pltpu.SemaphoreType.DMA,
        pltpu.SemaphoreType.DMA,
        pltpu.SemaphoreType.DMA((_NCHUNK,)),
        pltpu.SemaphoreType.DMA((2,)),
    ],
    compiler_params=pltpu.CompilerParams(use_tc_tiling_on_sc=False),
)


@jax.jit
def kernel(id, age, pvalue, shop, occu, city, gender, cms,
           W_id, W_age, W_pvalue, W_shop, W_occu, W_city, W_gender, W_cms):
    w_small = jnp.concatenate(
        [W_age, W_pvalue, W_shop, W_occu, W_city, W_gender, W_cms], axis=0)
    w_id_lin = lax.optimization_barrier(W_id.reshape(-1)).reshape(W_id.shape)
    out = _emb(id, age, pvalue, shop, occu, city, gender, cms, w_id_lin,
               w_small)
    return out.reshape(B, F, E)


# speedup vs baseline: 1.2836x; 1.0011x over previous
import jax
import jax.numpy as jnp
from jax import lax
from jax.experimental import pallas as pl
from jax.experimental.pallas import tpu as pltpu
from jax.experimental.pallas import tpu_sc as plsc

B = 16384
E = 64
F = 8

_NC = 2
_NS = 16
_NW = _NC * _NS
_BPW = B // _NW
_NCHUNK = 8
_CH = _BPW // _NCHUNK

_SMALL_OFFS = (0, 7, 11, 14, 16, 21, 23)
_SMALL_ROWS = 36


def _emb_body(id_h, age_h, pvalue_h, shop_h, occu_h, city_h, gender_h, cms_h,
              w_id_h, w_small_h, out_h,
              idx_v, ws_v, idbuf_v, asm_v, spidx_sh, sidx_m,
              isem, ssem, gsems, wsems):
    cid = lax.axis_index("c")
    sid = lax.axis_index("s")
    wid = sid * _NC + cid
    base = wid * _BPW
    idx_hbm = (age_h, pvalue_h, shop_h, occu_h, city_h, gender_h, cms_h)

    pltpu.sync_copy(id_h.at[pl.ds(base, _BPW)], idx_v.at[F - 1])

    gd = [pltpu.async_copy(w_id_h.at[idx_v.at[F - 1, pl.ds(k * _CH, _CH)]],
                           idbuf_v.at[k], gsems.at[k])
          for k in range(_NCHUNK)]

    icopies = [pltpu.async_copy(idx_hbm[f].at[pl.ds(base, _BPW)],
                                idx_v.at[f], isem) for f in range(F - 1)]
    scopy = pltpu.async_copy(w_small_h, ws_v, ssem)
    for c in icopies:
        c.wait()
    scopy.wait()
    bcopies = [pltpu.async_copy(idx_v.at[f, pl.ds(k * _CH, _CH)],
                                spidx_sh.at[sid, k, f], isem)
               for k in range(_NCHUNK) for f in range(F - 1)]
    for c in bcopies:
        c.wait()

    wd = [None] * _NCHUNK
    for k in range(_NCHUNK):
        p = k % 2
        pltpu.sync_copy(spidx_sh.at[sid, k], sidx_m)
        if k >= 2:
            wd[k - 2].wait()
        gd[k].wait()
        asm_p = asm_v.at[p]

        def row_body(i, _, asm_p=asm_p, k=k):
            for c in range(E // 16):
                asm_p[i, pl.ds(c * 16, 16)] = idbuf_v[k, i, pl.ds(c * 16, 16)]
            for f in range(F - 1):
                s = lax.min(lax.max(sidx_m[f, i], 0),
                            _SMALL_ROWS - 1 - _SMALL_OFFS[f]) + _SMALL_OFFS[f]
                for c in range(E // 16):
                    asm_p[i, pl.ds((f + 1) * E + c * 16, 16)] = \
                        ws_v[s, pl.ds(c * 16, 16)]
            return _

        lax.fori_loop(0, _CH, row_body, None)
        wd[k] = pltpu.async_copy(asm_p,
                                 out_h.at[pl.ds(base + k * _CH, _CH)],
                                 wsems.at[p])
    wd[_NCHUNK - 2].wait()
    wd[_NCHUNK - 1].wait()


_emb = pl.kernel(
    _emb_body,
    mesh=plsc.VectorSubcoreMesh(core_axis_name="c", subcore_axis_name="s"),
    out_type=jax.ShapeDtypeStruct((B, F * E), jnp.float32),
    scratch_types=[
        pltpu.VMEM((F, _BPW), jnp.int32),
        pltpu.VMEM((_SMALL_ROWS, E), jnp.float32),
        pltpu.VMEM((_NCHUNK, _CH, E), jnp.float32),
        pltpu.VMEM((2, _CH, F * E), jnp.float32),
        pltpu.VMEM_SHARED((_NS, _NCHUNK, F - 1, _CH), jnp.int32),
        pltpu.SMEM((F - 1, _CH), jnp.int32),
        pltpu.SemaphoreType.DMA,
        pltpu.SemaphoreType.DMA,
        pltpu.SemaphoreType.DMA((_NCHUNK,)),
        pltpu.SemaphoreType.DMA((2,)),
    ],
    compiler_params=pltpu.CompilerParams(use_tc_tiling_on_sc=False),
)


@jax.jit
def kernel(id, age, pvalue, shop, occu, city, gender, cms,
           W_id, W_age, W_pvalue, W_shop, W_occu, W_city, W_gender, W_cms):
    w_small = jnp.concatenate(
        [W_age, W_pvalue, W_shop, W_occu, W_city, W_gender, W_cms], axis=0)
    out = _emb(id, age, pvalue, shop, occu, city, gender, cms, W_id, w_small)
    return out.reshape(B, F, E)
